# baseline (speedup 1.0000x reference)
import functools

import jax
import jax.numpy as jnp
from jax import lax
from jax.experimental import pallas as pl
from jax.experimental.pallas import tpu as pltpu
from jax.experimental.pallas import tpu_sc as plsc

N = 10000
E = 320000
D = 128
DH = 64
H2 = 64

NC = 2
NS = 16
NPAD = 10240
RPT = NPAD // NS
C = 128
EPAD = NS * 20480
K = 80
NPH = 2
GC = 256
ERG = E // GC
DRG = (EPAD - E) // GC
KG = 40
UNROLL = 2
WCH = RPT // C


def _sc_agg_body(with_deg, x_lo, x_hi, edge_h, dsrc_h, ddst_h, zrows_h,
                 zrow1_h, ones_h, *refs):
    if with_deg:
        aglo_o, aghi_o, deg_o = refs[0], refs[1], refs[2]
        (src_v, dst_v, r0, r1, r2, r3, zbuf, ones_v, zbuf1,
         acc, dega, s0, s1, s2, s3, sdeg) = refs[3:]
    else:
        aglo_o, aghi_o = refs[0], refs[1]
        deg_o = None
        (src_v, dst_v, r0, r1, r2, r3, zbuf,
         acc, s0, s1, s2, s3) = refs[2:]
        sdeg = None
    c = lax.axis_index("c")
    s = lax.axis_index("s")

    pltpu.sync_copy(zrows_h, zbuf)
    if with_deg:
        pltpu.sync_copy(zrow1_h, zbuf1)
        pltpu.sync_copy(ones_h, ones_v)
    for j in range(WCH):
        pltpu.sync_copy(zbuf, acc.at[pl.ds(s * RPT + j * C, C)])
        if with_deg:
            pltpu.sync_copy(zbuf1, dega.at[pl.ds(s * RPT + j * C, C)])
    plsc.subcore_barrier()

    bufs = (r0, r1, r2, r3)
    sems = (s0, s1, s2, s3)

    def run_phase(x_h, count_deg):
        def step(t, carry):
            k0 = t * UNROLL
            gets = []
            for u in range(UNROLL):
                gets.append(pltpu.async_copy(
                    x_h.at[src_v.at[k0 + u]], bufs[u], sems[u]))
            puts = []
            for u in range(UNROLL):
                gets[u].wait()
                puts.append(pltpu.async_copy(
                    bufs[u], acc.at[dst_v.at[k0 + u]], sems[u],
                    add=True))
                if count_deg:
                    puts.append(pltpu.async_copy(
                        ones_v, dega.at[dst_v.at[k0 + u]], sdeg,
                        add=True))
            for p in puts:
                p.wait()
            return carry
        lax.fori_loop(0, KG // UNROLL, step, 0)

    for phase in range(NPH):
        base = s * (NPH * KG) + phase * KG
        if phase == 0:
            pltpu.sync_copy(edge_h.at[0, pl.ds(base, KG)], src_v)
            pltpu.sync_copy(edge_h.at[1, pl.ds(base, KG)], dst_v)
        else:
            @pl.when(s < NS - 1)
            def _():
                pltpu.sync_copy(edge_h.at[0, pl.ds(base, KG)], src_v)
                pltpu.sync_copy(edge_h.at[1, pl.ds(base, KG)], dst_v)

            @pl.when(s == NS - 1)
            def _():
                pltpu.sync_copy(
                    edge_h.at[0, pl.ds(ERG - (KG - DRG), KG - DRG)],
                    src_v.at[pl.ds(0, KG - DRG)])
                pltpu.sync_copy(
                    edge_h.at[1, pl.ds(ERG - (KG - DRG), KG - DRG)],
                    dst_v.at[pl.ds(0, KG - DRG)])
                pltpu.sync_copy(dsrc_h, src_v.at[pl.ds(KG - DRG, DRG)])
                pltpu.sync_copy(ddst_h, dst_v.at[pl.ds(KG - DRG, DRG)])

        @pl.when(c == 0)
        def _():
            run_phase(x_lo, with_deg)

        @pl.when(c == 1)
        def _():
            run_phase(x_hi, False)

    plsc.subcore_barrier()

    for j in range(WCH):
        sl = pl.ds(s * RPT + j * C, C)

        @pl.when(c == 0)
        def _():
            pltpu.sync_copy(acc.at[sl], r2)
            pltpu.sync_copy(r2, aglo_o.at[sl])

        @pl.when(c == 1)
        def _():
            pltpu.sync_copy(acc.at[sl], r3)
            pltpu.sync_copy(r3, aghi_o.at[sl])

        if with_deg:
            @pl.when(c == 0)
            def _():
                pltpu.sync_copy(dega.at[sl], zbuf1)
                pltpu.sync_copy(zbuf1, deg_o.at[sl])


@functools.cache
def _make_sc_agg(with_deg):
    out_type = [jax.ShapeDtypeStruct((NPAD, DH), jnp.float32),
                jax.ShapeDtypeStruct((NPAD, DH), jnp.float32)]
    if with_deg:
        out_type.append(jax.ShapeDtypeStruct((NPAD,), jnp.float32))
    mesh = plsc.VectorSubcoreMesh(
        core_axis_name="c", subcore_axis_name="s",
        num_cores=NC, num_subcores=NS)
    scratch = [
        pltpu.VMEM((KG, GC), jnp.int32),
        pltpu.VMEM((KG, GC), jnp.int32),
        pltpu.VMEM((GC, DH), jnp.float32),
        pltpu.VMEM((GC, DH), jnp.float32),
        pltpu.VMEM((C, DH), jnp.float32),
        pltpu.VMEM((C, DH), jnp.float32),
        pltpu.VMEM((C, DH), jnp.float32),
    ]
    if with_deg:
        scratch += [
            pltpu.VMEM((GC,), jnp.float32),
            pltpu.VMEM((C,), jnp.float32),
        ]
    scratch += [pltpu.VMEM_SHARED((NPAD, DH), jnp.float32)]
    if with_deg:
        scratch += [pltpu.VMEM_SHARED((NPAD,), jnp.float32)]
    scratch += [pltpu.SemaphoreType.DMA] * (5 if with_deg else 4)
    return pl.kernel(
        functools.partial(_sc_agg_body, with_deg),
        out_type=out_type, mesh=mesh, scratch_types=scratch,
        compiler_params=pltpu.CompilerParams(use_tc_tiling_on_sc=False))


_MM = functools.partial(jnp.dot, preferred_element_type=jnp.float32,
                        precision=lax.Precision.HIGHEST)


def _tc_self0(x_ref, ws_ref, b_ref, o_ref):
    o_ref[...] = _MM(x_ref[...], ws_ref[...]) + b_ref[...]


def _tc_neigh0(xw_ref, alo_ref, ahi_ref, dg_ref, wn_ref,
               olo_ref, ohi_ref):
    inv = 1.0 / jnp.maximum(dg_ref[...], 1.0)
    hn = jnp.concatenate([alo_ref[...] * inv, ahi_ref[...] * inv], axis=1)
    h = jnp.maximum(xw_ref[...] + _MM(hn, wn_ref[...]), 0.0)
    olo_ref[...] = h[:, :DH]
    ohi_ref[...] = h[:, DH:]


def _tc_self12(hlo_ref, hhi_ref, ws1_ref, b1_ref, ws2_ref, b2_ref,
               m_ref, l_ref):
    h = jnp.concatenate([hlo_ref[...], hhi_ref[...]], axis=1)
    m_ref[...] = _MM(h, ws1_ref[...]) + b1_ref[...]
    l_ref[...] = _MM(h, ws2_ref[...]) + b2_ref[...]


def _tc_final(m_ref, l_ref, alo_ref, ahi_ref, dg_ref, wn1_ref, wn2_ref,
              noise_ref, o_ref):
    inv = 1.0 / jnp.maximum(dg_ref[...], 1.0)
    hn = jnp.concatenate([alo_ref[...] * inv, ahi_ref[...] * inv], axis=1)
    mean = m_ref[...] + _MM(hn, wn1_ref[...])
    log_std = l_ref[...] + _MM(hn, wn2_ref[...])
    o_ref[...] = mean + noise_ref[...] * jnp.exp(log_std)


def kernel(features, edge_index, W_self0, W_neigh0, b0,
           W_self1, W_neigh1, b1, W_self2, W_neigh2, b2):
    pad = EPAD - E
    ar = jnp.arange(pad, dtype=jnp.int32)
    dummy_src = (ar % N).reshape(DRG, GC)
    dummy_dst = (N + ar % (NPAD - N)).reshape(DRG, GC)
    edge3d = edge_index.reshape(2, ERG, GC)

    zeros_rows = jnp.zeros((C, DH), jnp.float32)
    zeros_row1 = jnp.zeros((C,), jnp.float32)
    ones_row = jnp.ones((GC,), jnp.float32)

    x_lo = features[:, :DH]
    x_hi = features[:, DH:]
    aglo, aghi, deg = _make_sc_agg(True)(
        x_lo, x_hi, edge3d, dummy_src, dummy_dst,
        zeros_rows, zeros_row1, ones_row)
    deg1 = deg.reshape(NPAD, 1)

    BN = 1000
    row_blk = lambda w: pl.BlockSpec((BN, w), lambda i: (i, 0))
    full_blk = lambda r, w: pl.BlockSpec((r, w), lambda i: (0, 0))

    xw = pl.pallas_call(
        _tc_self0,
        grid=(N // BN,),
        in_specs=[row_blk(D), full_blk(D, D), full_blk(1, D)],
        out_specs=row_blk(D),
        out_shape=jax.ShapeDtypeStruct((N, D), jnp.float32),
    )(features, W_self0, b0[None, :])

    h_lo, h_hi = pl.pallas_call(
        _tc_neigh0,
        grid=(N // BN,),
        in_specs=[row_blk(D), row_blk(DH), row_blk(DH), row_blk(1),
                  full_blk(D, D)],
        out_specs=[row_blk(DH), row_blk(DH)],
        out_shape=[jax.ShapeDtypeStruct((N, DH), jnp.float32),
                   jax.ShapeDtypeStruct((N, DH), jnp.float32)],
    )(xw, aglo, aghi, deg1, W_neigh0)

    ahlo, ahhi = _make_sc_agg(False)(
        h_lo, h_hi, edge3d, dummy_src, dummy_dst,
        zeros_rows, zeros_row1, ones_row)

    m_self, l_self = pl.pallas_call(
        _tc_self12,
        grid=(N // BN,),
        in_specs=[row_blk(DH), row_blk(DH), full_blk(D, H2),
                  full_blk(1, H2), full_blk(D, H2), full_blk(1, H2)],
        out_specs=[row_blk(H2), row_blk(H2)],
        out_shape=[jax.ShapeDtypeStruct((N, H2), jnp.float32),
                   jax.ShapeDtypeStruct((N, H2), jnp.float32)],
    )(h_lo, h_hi, W_self1, b1[None, :], W_self2, b2[None, :])

    noise = jax.random.normal(jax.random.key(1), (N, H2), dtype=jnp.float32)
    z = pl.pallas_call(
        _tc_final,
        grid=(N // BN,),
        in_specs=[row_blk(H2), row_blk(H2), row_blk(DH), row_blk(DH),
                  row_blk(1), full_blk(D, H2), full_blk(D, H2),
                  row_blk(H2)],
        out_specs=pl.BlockSpec((BN, H2), lambda i: (i, 0)),
        out_shape=jax.ShapeDtypeStruct((N, H2), jnp.float32),
    )(m_self, l_self, ahlo, ahhi, deg1, W_neigh1, W_neigh2, noise)
    return z


# speedup vs baseline: 1.3781x; 1.3781x over previous
import functools

import jax
import jax.numpy as jnp
from jax import lax
from jax.experimental import pallas as pl
from jax.experimental.pallas import tpu as pltpu
from jax.experimental.pallas import tpu_sc as plsc

N = 10000
E = 320000
D = 128
DH = 64
H2 = 64

NC = 2
NS = 16
NPAD = 10240
RPT = NPAD // NS
C = 128
EPAD = NS * 20480
K = 80
NPH = 2
GC = 128
ERG = E // GC
DRG = (EPAD - E) // GC
KG = 80
UNROLL = 4
WCH = RPT // C


def _sc_agg_body(with_deg, x_lo, x_hi, edge_h, dsrc_h, ddst_h, zrows_h,
                 zrow1_h, ones_h, *refs):
    if with_deg:
        aglo_o, aghi_o, deg_o = refs[0], refs[1], refs[2]
        (src_v, dst_v, r0, r1, r2, r3, zbuf, ones_v, zbuf1,
         acc, dega, s0, s1, s2, s3, sdeg) = refs[3:]
    else:
        aglo_o, aghi_o = refs[0], refs[1]
        deg_o = None
        (src_v, dst_v, r0, r1, r2, r3, zbuf,
         acc, s0, s1, s2, s3) = refs[2:]
        sdeg = None
    c = lax.axis_index("c")
    s = lax.axis_index("s")

    pltpu.sync_copy(zrows_h, zbuf)
    if with_deg:
        pltpu.sync_copy(zrow1_h, zbuf1)
        pltpu.sync_copy(ones_h, ones_v)
    for j in range(WCH):
        pltpu.sync_copy(zbuf, acc.at[pl.ds(s * RPT + j * C, C)])
        if with_deg:
            pltpu.sync_copy(zbuf1, dega.at[pl.ds(s * RPT + j * C, C)])
    plsc.subcore_barrier()

    bufs = (r0, r1, r2, r3)
    sems = (s0, s1, s2, s3)

    def run_phase(x_h, count_deg):
        def step(t, carry):
            k0 = t * UNROLL
            gets = []
            for u in range(UNROLL):
                gets.append(pltpu.async_copy(
                    x_h.at[src_v.at[k0 + u]], bufs[u], sems[u]))
            puts = []
            for u in range(UNROLL):
                gets[u].wait()
                puts.append(pltpu.async_copy(
                    bufs[u], acc.at[dst_v.at[k0 + u]], sems[u],
                    add=True))
                if count_deg:
                    puts.append(pltpu.async_copy(
                        ones_v, dega.at[dst_v.at[k0 + u]], sdeg,
                        add=True))
            for p in puts:
                p.wait()
            return carry
        lax.fori_loop(0, KG // UNROLL, step, 0)

    for phase in range(NPH):
        base = s * (NPH * KG) + phase * KG
        if phase == 0:
            pltpu.sync_copy(edge_h.at[0, pl.ds(base, KG)], src_v)
            pltpu.sync_copy(edge_h.at[1, pl.ds(base, KG)], dst_v)
        else:
            @pl.when(s < NS - 1)
            def _():
                pltpu.sync_copy(edge_h.at[0, pl.ds(base, KG)], src_v)
                pltpu.sync_copy(edge_h.at[1, pl.ds(base, KG)], dst_v)

            @pl.when(s == NS - 1)
            def _():
                pltpu.sync_copy(
                    edge_h.at[0, pl.ds(ERG - (KG - DRG), KG - DRG)],
                    src_v.at[pl.ds(0, KG - DRG)])
                pltpu.sync_copy(
                    edge_h.at[1, pl.ds(ERG - (KG - DRG), KG - DRG)],
                    dst_v.at[pl.ds(0, KG - DRG)])
                pltpu.sync_copy(dsrc_h, src_v.at[pl.ds(KG - DRG, DRG)])
                pltpu.sync_copy(ddst_h, dst_v.at[pl.ds(KG - DRG, DRG)])

        @pl.when(c == 0)
        def _():
            run_phase(x_lo, with_deg)

        @pl.when(c == 1)
        def _():
            run_phase(x_hi, False)

    plsc.subcore_barrier()

    for j in range(WCH):
        sl = pl.ds(s * RPT + j * C, C)

        @pl.when(c == 0)
        def _():
            pltpu.sync_copy(acc.at[sl], r2)
            pltpu.sync_copy(r2, aglo_o.at[sl])

        @pl.when(c == 1)
        def _():
            pltpu.sync_copy(acc.at[sl], r3)
            pltpu.sync_copy(r3, aghi_o.at[sl])

        if with_deg:
            @pl.when(c == 0)
            def _():
                pltpu.sync_copy(dega.at[sl], zbuf1)
                pltpu.sync_copy(zbuf1, deg_o.at[sl])


@functools.cache
def _make_sc_agg(with_deg):
    out_type = [jax.ShapeDtypeStruct((NPAD, DH), jnp.bfloat16),
                jax.ShapeDtypeStruct((NPAD, DH), jnp.bfloat16)]
    if with_deg:
        out_type.append(jax.ShapeDtypeStruct((NPAD,), jnp.float32))
    mesh = plsc.VectorSubcoreMesh(
        core_axis_name="c", subcore_axis_name="s",
        num_cores=NC, num_subcores=NS)
    scratch = [
        pltpu.VMEM((KG, GC), jnp.int32),
        pltpu.VMEM((KG, GC), jnp.int32),
        pltpu.VMEM((GC, DH), jnp.bfloat16),
        pltpu.VMEM((GC, DH), jnp.bfloat16),
        pltpu.VMEM((GC, DH), jnp.bfloat16),
        pltpu.VMEM((GC, DH), jnp.bfloat16),
        pltpu.VMEM((C, DH), jnp.bfloat16),
    ]
    if with_deg:
        scratch += [
            pltpu.VMEM((GC,), jnp.float32),
            pltpu.VMEM((C,), jnp.float32),
        ]
    scratch += [pltpu.VMEM_SHARED((NPAD, DH), jnp.bfloat16)]
    if with_deg:
        scratch += [pltpu.VMEM_SHARED((NPAD,), jnp.float32)]
    scratch += [pltpu.SemaphoreType.DMA] * (5 if with_deg else 4)
    return pl.kernel(
        functools.partial(_sc_agg_body, with_deg),
        out_type=out_type, mesh=mesh, scratch_types=scratch,
        compiler_params=pltpu.CompilerParams(use_tc_tiling_on_sc=False))


_MM = functools.partial(jnp.dot, preferred_element_type=jnp.float32,
                        precision=lax.Precision.HIGHEST)


def _tc_self0(x_ref, ws_ref, b_ref, o_ref):
    o_ref[...] = _MM(x_ref[...], ws_ref[...]) + b_ref[...]


def _tc_neigh0(xw_ref, alo_ref, ahi_ref, dg_ref, wn_ref,
               olo_ref, ohi_ref, flo_ref, fhi_ref):
    inv = 1.0 / jnp.maximum(dg_ref[...], 1.0)
    hn = jnp.concatenate(
        [alo_ref[...].astype(jnp.float32) * inv,
         ahi_ref[...].astype(jnp.float32) * inv], axis=1)
    h = jnp.maximum(xw_ref[...] + _MM(hn, wn_ref[...]), 0.0)
    hb = h.astype(jnp.bfloat16)
    olo_ref[...] = hb[:, :DH]
    ohi_ref[...] = hb[:, DH:]
    flo_ref[...] = h[:, :DH]
    fhi_ref[...] = h[:, DH:]


def _tc_self12(hlo_ref, hhi_ref, ws1_ref, b1_ref, ws2_ref, b2_ref,
               m_ref, l_ref):
    h = jnp.concatenate([hlo_ref[...], hhi_ref[...]], axis=1)
    m_ref[...] = _MM(h, ws1_ref[...]) + b1_ref[...]
    l_ref[...] = _MM(h, ws2_ref[...]) + b2_ref[...]


def _tc_final(m_ref, l_ref, alo_ref, ahi_ref, dg_ref, wn1_ref, wn2_ref,
              noise_ref, o_ref):
    inv = 1.0 / jnp.maximum(dg_ref[...], 1.0)
    hn = jnp.concatenate(
        [alo_ref[...].astype(jnp.float32) * inv,
         ahi_ref[...].astype(jnp.float32) * inv], axis=1)
    mean = m_ref[...] + _MM(hn, wn1_ref[...])
    log_std = l_ref[...] + _MM(hn, wn2_ref[...])
    o_ref[...] = mean + noise_ref[...] * jnp.exp(log_std)


def kernel(features, edge_index, W_self0, W_neigh0, b0,
           W_self1, W_neigh1, b1, W_self2, W_neigh2, b2):
    pad = EPAD - E
    ar = jnp.arange(pad, dtype=jnp.int32)
    dummy_src = (ar % N).reshape(DRG, GC)
    dummy_dst = (N + ar % (NPAD - N)).reshape(DRG, GC)
    edge3d = edge_index.reshape(2, ERG, GC)

    zeros_rows = jnp.zeros((C, DH), jnp.bfloat16)
    zeros_row1 = jnp.zeros((C,), jnp.float32)
    ones_row = jnp.ones((GC,), jnp.float32)

    xb = features.astype(jnp.bfloat16)
    x_lo = xb[:, :DH]
    x_hi = xb[:, DH:]
    aglo, aghi, deg = _make_sc_agg(True)(
        x_lo, x_hi, edge3d, dummy_src, dummy_dst,
        zeros_rows, zeros_row1, ones_row)
    deg1 = deg.reshape(NPAD, 1)

    BN = 1000
    row_blk = lambda w: pl.BlockSpec((BN, w), lambda i: (i, 0))
    full_blk = lambda r, w: pl.BlockSpec((r, w), lambda i: (0, 0))

    xw = pl.pallas_call(
        _tc_self0,
        grid=(N // BN,),
        in_specs=[row_blk(D), full_blk(D, D), full_blk(1, D)],
        out_specs=row_blk(D),
        out_shape=jax.ShapeDtypeStruct((N, D), jnp.float32),
    )(features, W_self0, b0[None, :])

    h_lo, h_hi, hf_lo, hf_hi = pl.pallas_call(
        _tc_neigh0,
        grid=(N // BN,),
        in_specs=[row_blk(D), row_blk(DH), row_blk(DH), row_blk(1),
                  full_blk(D, D)],
        out_specs=[row_blk(DH), row_blk(DH), row_blk(DH), row_blk(DH)],
        out_shape=[jax.ShapeDtypeStruct((N, DH), jnp.bfloat16),
                   jax.ShapeDtypeStruct((N, DH), jnp.bfloat16),
                   jax.ShapeDtypeStruct((N, DH), jnp.float32),
                   jax.ShapeDtypeStruct((N, DH), jnp.float32)],
    )(xw, aglo, aghi, deg1, W_neigh0)

    ahlo, ahhi = _make_sc_agg(False)(
        h_lo, h_hi, edge3d, dummy_src, dummy_dst,
        zeros_rows, zeros_row1, ones_row)

    m_self, l_self = pl.pallas_call(
        _tc_self12,
        grid=(N // BN,),
        in_specs=[row_blk(DH), row_blk(DH), full_blk(D, H2),
                  full_blk(1, H2), full_blk(D, H2), full_blk(1, H2)],
        out_specs=[row_blk(H2), row_blk(H2)],
        out_shape=[jax.ShapeDtypeStruct((N, H2), jnp.float32),
                   jax.ShapeDtypeStruct((N, H2), jnp.float32)],
    )(hf_lo, hf_hi, W_self1, b1[None, :], W_self2, b2[None, :])

    noise = jax.random.normal(jax.random.key(1), (N, H2), dtype=jnp.float32)
    z = pl.pallas_call(
        _tc_final,
        grid=(N // BN,),
        in_specs=[row_blk(H2), row_blk(H2), row_blk(DH), row_blk(DH),
                  row_blk(1), full_blk(D, H2), full_blk(D, H2),
                  row_blk(H2)],
        out_specs=pl.BlockSpec((BN, H2), lambda i: (i, 0)),
        out_shape=jax.ShapeDtypeStruct((N, H2), jnp.float32),
    )(m_self, l_self, ahlo, ahhi, deg1, W_neigh1, W_neigh2, noise)
    return z


# speedup vs baseline: 1.4394x; 1.0445x over previous
import functools

import jax
import jax.numpy as jnp
from jax import lax
from jax.experimental import pallas as pl
from jax.experimental.pallas import tpu as pltpu
from jax.experimental.pallas import tpu_sc as plsc

N = 10000
E = 320000
D = 128
DH = 64
H2 = 64

NC = 2
NS = 16
NPAD = 10240
RPT = NPAD // NS
C = 128
EPAD = NS * 20480
K = 80
NPH = 2
GC = 128
ERG = E // GC
DRG = (EPAD - E) // GC
KG = 80
UNROLL = 4
WCH = RPT // C


def _sc_agg_body(with_deg, x_lo, x_hi, edge_h, dsrc_h, ddst_h, zrows_h,
                 zrow1_h, ones_h, *refs):
    if with_deg:
        aglo_o, aghi_o, deg_o = refs[0], refs[1], refs[2]
        (src_v, dst_v, r0, r1, r2, r3, zbuf, ones_v, zbuf1,
         acc, dega, s0, s1, s2, s3, sdeg) = refs[3:]
    else:
        aglo_o, aghi_o = refs[0], refs[1]
        deg_o = None
        (src_v, dst_v, r0, r1, r2, r3, zbuf,
         acc, s0, s1, s2, s3) = refs[2:]
        sdeg = None
    c = lax.axis_index("c")
    s = lax.axis_index("s")

    pltpu.sync_copy(zrows_h, zbuf)
    if with_deg:
        pltpu.sync_copy(zrow1_h, zbuf1)
        pltpu.sync_copy(ones_h, ones_v)
    for j in range(WCH):
        pltpu.sync_copy(zbuf, acc.at[pl.ds(s * RPT + j * C, C)])
        if with_deg:
            pltpu.sync_copy(zbuf1, dega.at[pl.ds(s * RPT + j * C, C)])
    plsc.subcore_barrier()

    bufs = (r0, r1, r2, r3)
    sems = (s0, s1, s2, s3)

    def run_phase(x_h, count_deg):
        def step(t, carry):
            k0 = t * UNROLL
            gets = []
            for u in range(UNROLL):
                gets.append(pltpu.async_copy(
                    x_h.at[src_v.at[k0 + u]], bufs[u], sems[u]))
            puts = []
            for u in range(UNROLL):
                gets[u].wait()
                puts.append(pltpu.async_copy(
                    bufs[u], acc.at[dst_v.at[k0 + u]], sems[u],
                    add=True))
                if count_deg:
                    puts.append(pltpu.async_copy(
                        ones_v, dega.at[dst_v.at[k0 + u]], sdeg,
                        add=True))
            for p in puts:
                p.wait()
            return carry
        lax.fori_loop(0, KG // UNROLL, step, 0)

    for phase in range(NPH):
        base = s * (NPH * KG) + phase * KG
        if phase == 0:
            pltpu.sync_copy(edge_h.at[0, pl.ds(base, KG)], src_v)
            pltpu.sync_copy(edge_h.at[1, pl.ds(base, KG)], dst_v)
        else:
            @pl.when(s < NS - 1)
            def _():
                pltpu.sync_copy(edge_h.at[0, pl.ds(base, KG)], src_v)
                pltpu.sync_copy(edge_h.at[1, pl.ds(base, KG)], dst_v)

            @pl.when(s == NS - 1)
            def _():
                pltpu.sync_copy(
                    edge_h.at[0, pl.ds(ERG - (KG - DRG), KG - DRG)],
                    src_v.at[pl.ds(0, KG - DRG)])
                pltpu.sync_copy(
                    edge_h.at[1, pl.ds(ERG - (KG - DRG), KG - DRG)],
                    dst_v.at[pl.ds(0, KG - DRG)])
                pltpu.sync_copy(dsrc_h, src_v.at[pl.ds(KG - DRG, DRG)])
                pltpu.sync_copy(ddst_h, dst_v.at[pl.ds(KG - DRG, DRG)])

        @pl.when(c == 0)
        def _():
            run_phase(x_lo, with_deg)

        @pl.when(c == 1)
        def _():
            run_phase(x_hi, False)

    plsc.subcore_barrier()

    for j in range(WCH):
        sl = pl.ds(s * RPT + j * C, C)

        @pl.when(c == 0)
        def _():
            pltpu.sync_copy(acc.at[sl], r2)
            pltpu.sync_copy(r2, aglo_o.at[sl])

        @pl.when(c == 1)
        def _():
            pltpu.sync_copy(acc.at[sl], r3)
            pltpu.sync_copy(r3, aghi_o.at[sl])

        if with_deg:
            @pl.when(c == 0)
            def _():
                pltpu.sync_copy(dega.at[sl], zbuf1)
                pltpu.sync_copy(zbuf1, deg_o.at[sl])


@functools.cache
def _make_sc_agg(with_deg):
    out_type = [jax.ShapeDtypeStruct((NPAD, DH), jnp.bfloat16),
                jax.ShapeDtypeStruct((NPAD, DH), jnp.bfloat16)]
    if with_deg:
        out_type.append(jax.ShapeDtypeStruct((NPAD,), jnp.float32))
    mesh = plsc.VectorSubcoreMesh(
        core_axis_name="c", subcore_axis_name="s",
        num_cores=NC, num_subcores=NS)
    scratch = [
        pltpu.VMEM((KG, GC), jnp.int32),
        pltpu.VMEM((KG, GC), jnp.int32),
        pltpu.VMEM((GC, DH), jnp.bfloat16),
        pltpu.VMEM((GC, DH), jnp.bfloat16),
        pltpu.VMEM((GC, DH), jnp.bfloat16),
        pltpu.VMEM((GC, DH), jnp.bfloat16),
        pltpu.VMEM((C, DH), jnp.bfloat16),
    ]
    if with_deg:
        scratch += [
            pltpu.VMEM((GC,), jnp.float32),
            pltpu.VMEM((C,), jnp.float32),
        ]
    scratch += [pltpu.VMEM_SHARED((NPAD, DH), jnp.bfloat16)]
    if with_deg:
        scratch += [pltpu.VMEM_SHARED((NPAD,), jnp.float32)]
    scratch += [pltpu.SemaphoreType.DMA] * (5 if with_deg else 4)
    return pl.kernel(
        functools.partial(_sc_agg_body, with_deg),
        out_type=out_type, mesh=mesh, scratch_types=scratch,
        compiler_params=pltpu.CompilerParams(use_tc_tiling_on_sc=False))


_MM = functools.partial(jnp.dot, preferred_element_type=jnp.float32,
                        precision=lax.Precision.HIGHEST)


def _tc_self0(x_ref, ws_ref, b_ref, o_ref):
    o_ref[...] = _MM(x_ref[...], ws_ref[...]) + b_ref[...]


def _tc_neigh0(xw_ref, alo_ref, ahi_ref, dg_ref, wn_ref,
               olo_ref, ohi_ref):
    inv = 1.0 / jnp.maximum(dg_ref[...], 1.0)
    hn = jnp.concatenate(
        [alo_ref[...].astype(jnp.float32) * inv,
         ahi_ref[...].astype(jnp.float32) * inv], axis=1)
    h = jnp.maximum(xw_ref[...] + _MM(hn, wn_ref[...]), 0.0)
    hb = h.astype(jnp.bfloat16)
    olo_ref[...] = hb[:, :DH]
    ohi_ref[...] = hb[:, DH:]


def _tc_self12(hlo_ref, hhi_ref, ws1_ref, b1_ref, ws2_ref, b2_ref,
               m_ref, l_ref):
    h = jnp.concatenate([hlo_ref[...], hhi_ref[...]],
                        axis=1).astype(jnp.float32)
    m_ref[...] = _MM(h, ws1_ref[...]) + b1_ref[...]
    l_ref[...] = _MM(h, ws2_ref[...]) + b2_ref[...]


def _tc_final(m_ref, l_ref, alo_ref, ahi_ref, dg_ref, wn1_ref, wn2_ref,
              noise_ref, o_ref):
    inv = 1.0 / jnp.maximum(dg_ref[...], 1.0)
    hn = jnp.concatenate(
        [alo_ref[...].astype(jnp.float32) * inv,
         ahi_ref[...].astype(jnp.float32) * inv], axis=1)
    mean = m_ref[...] + _MM(hn, wn1_ref[...])
    log_std = l_ref[...] + _MM(hn, wn2_ref[...])
    o_ref[...] = mean + noise_ref[...] * jnp.exp(log_std)


def kernel(features, edge_index, W_self0, W_neigh0, b0,
           W_self1, W_neigh1, b1, W_self2, W_neigh2, b2):
    pad = EPAD - E
    ar = jnp.arange(pad, dtype=jnp.int32)
    dummy_src = (ar % N).reshape(DRG, GC)
    dummy_dst = (N + ar % (NPAD - N)).reshape(DRG, GC)
    edge3d = edge_index.reshape(2, ERG, GC)

    zeros_rows = jnp.zeros((C, DH), jnp.bfloat16)
    zeros_row1 = jnp.zeros((C,), jnp.float32)
    ones_row = jnp.ones((GC,), jnp.float32)

    xb = features.astype(jnp.bfloat16)
    x_lo = xb[:, :DH]
    x_hi = xb[:, DH:]
    aglo, aghi, deg = _make_sc_agg(True)(
        x_lo, x_hi, edge3d, dummy_src, dummy_dst,
        zeros_rows, zeros_row1, ones_row)
    deg1 = deg.reshape(NPAD, 1)

    BN = 2000
    row_blk = lambda w: pl.BlockSpec((BN, w), lambda i: (i, 0))
    full_blk = lambda r, w: pl.BlockSpec((r, w), lambda i: (0, 0))

    xw = pl.pallas_call(
        _tc_self0,
        grid=(N // BN,),
        in_specs=[row_blk(D), full_blk(D, D), full_blk(1, D)],
        out_specs=row_blk(D),
        out_shape=jax.ShapeDtypeStruct((N, D), jnp.float32),
    )(features, W_self0, b0[None, :])

    h_lo, h_hi = pl.pallas_call(
        _tc_neigh0,
        grid=(N // BN,),
        in_specs=[row_blk(D), row_blk(DH), row_blk(DH), row_blk(1),
                  full_blk(D, D)],
        out_specs=[row_blk(DH), row_blk(DH)],
        out_shape=[jax.ShapeDtypeStruct((N, DH), jnp.bfloat16),
                   jax.ShapeDtypeStruct((N, DH), jnp.bfloat16)],
    )(xw, aglo, aghi, deg1, W_neigh0)

    ahlo, ahhi = _make_sc_agg(False)(
        h_lo, h_hi, edge3d, dummy_src, dummy_dst,
        zeros_rows, zeros_row1, ones_row)

    m_self, l_self = pl.pallas_call(
        _tc_self12,
        grid=(N // BN,),
        in_specs=[row_blk(DH), row_blk(DH), full_blk(D, H2),
                  full_blk(1, H2), full_blk(D, H2), full_blk(1, H2)],
        out_specs=[row_blk(H2), row_blk(H2)],
        out_shape=[jax.ShapeDtypeStruct((N, H2), jnp.float32),
                   jax.ShapeDtypeStruct((N, H2), jnp.float32)],
    )(h_lo, h_hi, W_self1, b1[None, :], W_self2, b2[None, :])

    noise = jax.random.normal(jax.random.key(1), (N, H2), dtype=jnp.float32)
    z = pl.pallas_call(
        _tc_final,
        grid=(N // BN,),
        in_specs=[row_blk(H2), row_blk(H2), row_blk(DH), row_blk(DH),
                  row_blk(1), full_blk(D, H2), full_blk(D, H2),
                  row_blk(H2)],
        out_specs=pl.BlockSpec((BN, H2), lambda i: (i, 0)),
        out_shape=jax.ShapeDtypeStruct((N, H2), jnp.float32),
    )(m_self, l_self, ahlo, ahhi, deg1, W_neigh1, W_neigh2, noise)
    return z


# speedup vs baseline: 1.4707x; 1.0217x over previous
import functools

import jax
import jax.numpy as jnp
from jax import lax
from jax.experimental import pallas as pl
from jax.experimental.pallas import tpu as pltpu
from jax.experimental.pallas import tpu_sc as plsc

N = 10000
E = 320000
D = 128
DH = 64
H2 = 64

NC = 2
NS = 16
NPAD = 10240
RPT = NPAD // NS
C = 128
EPAD = NS * 20480
K = 80
NPH = 2
GC = 128
ERG = E // GC
DRG = (EPAD - E) // GC
KG = 80
UNROLL = 6
WCH = RPT // C


def _sc_agg_body(with_deg, x_lo, x_hi, edge_h, dsrc_h, ddst_h, zrows_h,
                 zrow1_h, ones_h, *refs):
    if with_deg:
        aglo_o, aghi_o, deg_o = refs[0], refs[1], refs[2]
        (src_v, dst_v, r0, r1, r2, r3, r4, r5, zbuf, ones_v, zbuf1,
         acc, dega, s0, s1, s2, s3, s4, s5, sdeg) = refs[3:]
    else:
        aglo_o, aghi_o = refs[0], refs[1]
        deg_o = None
        (src_v, dst_v, r0, r1, r2, r3, r4, r5, zbuf,
         acc, s0, s1, s2, s3, s4, s5) = refs[2:]
        sdeg = None
    c = lax.axis_index("c")
    s = lax.axis_index("s")

    pltpu.sync_copy(zrows_h, zbuf)
    if with_deg:
        pltpu.sync_copy(zrow1_h, zbuf1)
        pltpu.sync_copy(ones_h, ones_v)
    for j in range(WCH):
        pltpu.sync_copy(zbuf, acc.at[pl.ds(s * RPT + j * C, C)])
        if with_deg:
            pltpu.sync_copy(zbuf1, dega.at[pl.ds(s * RPT + j * C, C)])
    plsc.subcore_barrier()

    bufs = (r0, r1, r2, r3, r4, r5)
    sems = (s0, s1, s2, s3, s4, s5)

    def run_phase(x_h, count_deg):
        def step(t, carry):
            k0 = t * UNROLL
            gets = []
            for u in range(UNROLL):
                gets.append(pltpu.async_copy(
                    x_h.at[src_v.at[k0 + u]], bufs[u], sems[u]))
            puts = []
            for u in range(UNROLL):
                gets[u].wait()
                puts.append(pltpu.async_copy(
                    bufs[u], acc.at[dst_v.at[k0 + u]], sems[u],
                    add=True))
                if count_deg:
                    puts.append(pltpu.async_copy(
                        ones_v, dega.at[dst_v.at[k0 + u]], sdeg,
                        add=True))
            for p in puts:
                p.wait()
            return carry
        lax.fori_loop(0, KG // UNROLL, step, 0)
        tail = KG - (KG // UNROLL) * UNROLL
        if tail:
            base_t = (KG // UNROLL) * UNROLL
            gets = []
            for u in range(tail):
                gets.append(pltpu.async_copy(
                    x_h.at[src_v.at[base_t + u]], bufs[u], sems[u]))
            puts = []
            for u in range(tail):
                gets[u].wait()
                puts.append(pltpu.async_copy(
                    bufs[u], acc.at[dst_v.at[base_t + u]], sems[u],
                    add=True))
                if count_deg:
                    puts.append(pltpu.async_copy(
                        ones_v, dega.at[dst_v.at[base_t + u]], sdeg,
                        add=True))
            for p in puts:
                p.wait()

    for phase in range(NPH):
        base = s * (NPH * KG) + phase * KG
        if phase == 0:
            pltpu.sync_copy(edge_h.at[0, pl.ds(base, KG)], src_v)
            pltpu.sync_copy(edge_h.at[1, pl.ds(base, KG)], dst_v)
        else:
            @pl.when(s < NS - 1)
            def _():
                pltpu.sync_copy(edge_h.at[0, pl.ds(base, KG)], src_v)
                pltpu.sync_copy(edge_h.at[1, pl.ds(base, KG)], dst_v)

            @pl.when(s == NS - 1)
            def _():
                pltpu.sync_copy(
                    edge_h.at[0, pl.ds(ERG - (KG - DRG), KG - DRG)],
                    src_v.at[pl.ds(0, KG - DRG)])
                pltpu.sync_copy(
                    edge_h.at[1, pl.ds(ERG - (KG - DRG), KG - DRG)],
                    dst_v.at[pl.ds(0, KG - DRG)])
                pltpu.sync_copy(dsrc_h, src_v.at[pl.ds(KG - DRG, DRG)])
                pltpu.sync_copy(ddst_h, dst_v.at[pl.ds(KG - DRG, DRG)])

        @pl.when(c == 0)
        def _():
            run_phase(x_lo, with_deg and phase == 0)

        @pl.when(c == 1)
        def _():
            run_phase(x_hi, with_deg and phase == 1)

    plsc.subcore_barrier()

    for j in range(WCH):
        sl = pl.ds(s * RPT + j * C, C)

        @pl.when(c == 0)
        def _():
            pltpu.sync_copy(acc.at[sl], r2)
            pltpu.sync_copy(r2, aglo_o.at[sl])

        @pl.when(c == 1)
        def _():
            pltpu.sync_copy(acc.at[sl], r3)
            pltpu.sync_copy(r3, aghi_o.at[sl])

        if with_deg:
            pltpu.sync_copy(dega.at[sl], zbuf1)
            pltpu.sync_copy(zbuf1, deg_o.at[c, sl])


@functools.cache
def _make_sc_agg(with_deg):
    out_type = [jax.ShapeDtypeStruct((NPAD, DH), jnp.bfloat16),
                jax.ShapeDtypeStruct((NPAD, DH), jnp.bfloat16)]
    if with_deg:
        out_type.append(jax.ShapeDtypeStruct((NC, NPAD), jnp.float32))
    mesh = plsc.VectorSubcoreMesh(
        core_axis_name="c", subcore_axis_name="s",
        num_cores=NC, num_subcores=NS)
    scratch = [
        pltpu.VMEM((KG, GC), jnp.int32),
        pltpu.VMEM((KG, GC), jnp.int32),
        pltpu.VMEM((GC, DH), jnp.bfloat16),
        pltpu.VMEM((GC, DH), jnp.bfloat16),
        pltpu.VMEM((GC, DH), jnp.bfloat16),
        pltpu.VMEM((GC, DH), jnp.bfloat16),
        pltpu.VMEM((GC, DH), jnp.bfloat16),
        pltpu.VMEM((GC, DH), jnp.bfloat16),
        pltpu.VMEM((C, DH), jnp.bfloat16),
    ]
    if with_deg:
        scratch += [
            pltpu.VMEM((GC,), jnp.float32),
            pltpu.VMEM((C,), jnp.float32),
        ]
    scratch += [pltpu.VMEM_SHARED((NPAD, DH), jnp.bfloat16)]
    if with_deg:
        scratch += [pltpu.VMEM_SHARED((NPAD,), jnp.float32)]
    scratch += [pltpu.SemaphoreType.DMA] * (7 if with_deg else 6)
    return pl.kernel(
        functools.partial(_sc_agg_body, with_deg),
        out_type=out_type, mesh=mesh, scratch_types=scratch,
        compiler_params=pltpu.CompilerParams(use_tc_tiling_on_sc=False))


_MM = functools.partial(jnp.dot, preferred_element_type=jnp.float32,
                        precision=lax.Precision.HIGHEST)


def _tc_self0(x_ref, ws_ref, b_ref, o_ref):
    o_ref[...] = _MM(x_ref[...], ws_ref[...]) + b_ref[...]


def _tc_neigh0(xw_ref, alo_ref, ahi_ref, dg0_ref, dg1_ref, wn_ref,
               olo_ref, ohi_ref):
    inv = 1.0 / jnp.maximum(dg0_ref[0] + dg1_ref[0], 1.0)
    hn = jnp.concatenate(
        [alo_ref[...].astype(jnp.float32) * inv,
         ahi_ref[...].astype(jnp.float32) * inv], axis=1)
    h = jnp.maximum(xw_ref[...] + _MM(hn, wn_ref[...]), 0.0)
    hb = h.astype(jnp.bfloat16)
    olo_ref[...] = hb[:, :DH]
    ohi_ref[...] = hb[:, DH:]


def _tc_self12(hlo_ref, hhi_ref, ws1_ref, b1_ref, ws2_ref, b2_ref,
               m_ref, l_ref):
    h = jnp.concatenate([hlo_ref[...], hhi_ref[...]],
                        axis=1).astype(jnp.float32)
    m_ref[...] = _MM(h, ws1_ref[...]) + b1_ref[...]
    l_ref[...] = _MM(h, ws2_ref[...]) + b2_ref[...]


def _tc_final(m_ref, l_ref, alo_ref, ahi_ref, dg0_ref, dg1_ref,
              wn1_ref, wn2_ref, noise_ref, o_ref):
    inv = 1.0 / jnp.maximum(dg0_ref[0] + dg1_ref[0], 1.0)
    hn = jnp.concatenate(
        [alo_ref[...].astype(jnp.float32) * inv,
         ahi_ref[...].astype(jnp.float32) * inv], axis=1)
    mean = m_ref[...] + _MM(hn, wn1_ref[...])
    log_std = l_ref[...] + _MM(hn, wn2_ref[...])
    o_ref[...] = mean + noise_ref[...] * jnp.exp(log_std)


def kernel(features, edge_index, W_self0, W_neigh0, b0,
           W_self1, W_neigh1, b1, W_self2, W_neigh2, b2):
    pad = EPAD - E
    ar = jnp.arange(pad, dtype=jnp.int32)
    dummy_src = (ar % N).reshape(DRG, GC)
    dummy_dst = (N + ar % (NPAD - N)).reshape(DRG, GC)
    edge3d = edge_index.reshape(2, ERG, GC)

    zeros_rows = jnp.zeros((C, DH), jnp.bfloat16)
    zeros_row1 = jnp.zeros((C,), jnp.float32)
    ones_row = jnp.ones((GC,), jnp.float32)

    xb = features.astype(jnp.bfloat16)
    x_lo = xb[:, :DH]
    x_hi = xb[:, DH:]
    aglo, aghi, degp = _make_sc_agg(True)(
        x_lo, x_hi, edge3d, dummy_src, dummy_dst,
        zeros_rows, zeros_row1, ones_row)
    deg3 = degp.reshape(NC, NPAD, 1)

    BN = 2000
    row_blk = lambda w: pl.BlockSpec((BN, w), lambda i: (i, 0))
    full_blk = lambda r, w: pl.BlockSpec((r, w), lambda i: (0, 0))

    xw = pl.pallas_call(
        _tc_self0,
        grid=(N // BN,),
        in_specs=[row_blk(D), full_blk(D, D), full_blk(1, D)],
        out_specs=row_blk(D),
        out_shape=jax.ShapeDtypeStruct((N, D), jnp.float32),
    )(features, W_self0, b0[None, :])

    h_lo, h_hi = pl.pallas_call(
        _tc_neigh0,
        grid=(N // BN,),
        in_specs=[row_blk(D), row_blk(DH), row_blk(DH),
                  pl.BlockSpec((1, BN, 1), lambda i: (0, i, 0)),
                  pl.BlockSpec((1, BN, 1), lambda i: (1, i, 0)),
                  full_blk(D, D)],
        out_specs=[row_blk(DH), row_blk(DH)],
        out_shape=[jax.ShapeDtypeStruct((N, DH), jnp.bfloat16),
                   jax.ShapeDtypeStruct((N, DH), jnp.bfloat16)],
    )(xw, aglo, aghi, deg3, deg3, W_neigh0)

    ahlo, ahhi = _make_sc_agg(False)(
        h_lo, h_hi, edge3d, dummy_src, dummy_dst,
        zeros_rows, zeros_row1, ones_row)

    m_self, l_self = pl.pallas_call(
        _tc_self12,
        grid=(N // BN,),
        in_specs=[row_blk(DH), row_blk(DH), full_blk(D, H2),
                  full_blk(1, H2), full_blk(D, H2), full_blk(1, H2)],
        out_specs=[row_blk(H2), row_blk(H2)],
        out_shape=[jax.ShapeDtypeStruct((N, H2), jnp.float32),
                   jax.ShapeDtypeStruct((N, H2), jnp.float32)],
    )(h_lo, h_hi, W_self1, b1[None, :], W_self2, b2[None, :])

    noise = jax.random.normal(jax.random.key(1), (N, H2), dtype=jnp.float32)
    z = pl.pallas_call(
        _tc_final,
        grid=(N // BN,),
        in_specs=[row_blk(H2), row_blk(H2), row_blk(DH), row_blk(DH),
                  pl.BlockSpec((1, BN, 1), lambda i: (0, i, 0)),
                  pl.BlockSpec((1, BN, 1), lambda i: (1, i, 0)),
                  full_blk(D, H2), full_blk(D, H2),
                  row_blk(H2)],
        out_specs=pl.BlockSpec((BN, H2), lambda i: (i, 0)),
        out_shape=jax.ShapeDtypeStruct((N, H2), jnp.float32),
    )(m_self, l_self, ahlo, ahhi, deg3, deg3, W_neigh1, W_neigh2, noise)
    return z
